# baseline (speedup 1.0000x reference)
import functools

import jax
import jax.numpy as jnp
from jax import lax
from jax.experimental import pallas as pl
from jax.experimental.pallas import tpu as pltpu
from jax.experimental.pallas import tpu_sc as plsc

_N = 10000
_E = 320000
_D = 128
_G = 256
_OUT = 768
_LAYERS = 4
_NODE_SIZES = [119, 9, 11, 12, 9, 5, 8, 2, 2]
_NODE_TOT = sum(_NODE_SIZES)
_NODE_PAD = 184
_EDGE_SIZES = [22, 6, 2]
_EDGE_TOT = sum(_EDGE_SIZES)
_EDGE_PAD = 32
_NCOMB = 264

_SC_CORES = 2
_SC_TILES = 16
_NWORK = _SC_CORES * _SC_TILES
_EPT = _E // _NWORK
_B = 80
_NCHT = 128
_NCHP = 126
_NG = _NCHP // 2
_EPAD = _NCHT * _B - _EPT
_NPAD = 10240
_RPT = _NPAD // _SC_TILES


def _cidx_body(ea_ref, out_ref):
    a0 = ea_ref[0]
    a1 = ea_ref[1]
    a2 = ea_ref[2]
    out_ref[...] = a0 * 12 + a1 * 2 + a2


def _compute_cidx(ea_r):
    return pl.pallas_call(
        _cidx_body,
        out_shape=jax.ShapeDtypeStruct((_E // 128, 128), jnp.int32),
    )(ea_r)


def _node_enc_body(xc_ref, tab_ref, out_ref):
    cols = lax.broadcasted_iota(jnp.int32, (xc_ref.shape[0], _NODE_PAD), 1)
    mh = jnp.zeros((xc_ref.shape[0], _NODE_PAD), jnp.float32)
    off = 0
    for j, sz in enumerate(_NODE_SIZES):
        idx = xc_ref[:, j][:, None] + off
        mh = mh + (cols == idx).astype(jnp.float32)
        off += sz
    out_ref[...] = jnp.dot(mh, tab_ref[...],
                           preferred_element_type=jnp.float32)


def _node_encode(x_cat, node_tab):
    blk = 1000
    return pl.pallas_call(
        _node_enc_body,
        grid=(_N // blk,),
        in_specs=[
            pl.BlockSpec((blk, 9), lambda i: (i, 0)),
            pl.BlockSpec((_NODE_PAD, _D), lambda i: (0, 0)),
        ],
        out_specs=pl.BlockSpec((blk, _D), lambda i: (i, 0)),
        out_shape=jax.ShapeDtypeStruct((_N, _D), jnp.float32),
    )(x_cat, node_tab)


def _tcomb_body(etab_ref, lw_ref, lb_ref, out_ref):
    rows = lax.broadcasted_iota(jnp.int32, (_NCOMB, _EDGE_PAD), 0)
    cols = lax.broadcasted_iota(jnp.int32, (_NCOMB, _EDGE_PAD), 1)
    c0 = rows // 12
    c1 = (rows % 12) // 2
    c2 = rows % 2
    sel = ((cols == c0) | (cols == c1 + 22) | (cols == c2 + 28))
    comb = jnp.dot(sel.astype(jnp.float32), etab_ref[...],
                   preferred_element_type=jnp.float32)
    for l in range(_LAYERS):
        out_ref[l] = (
            jnp.dot(comb, lw_ref[l], preferred_element_type=jnp.float32)
            + lb_ref[l][None, :]
        )


def _build_tcomb(etab, lw, lb):
    return pl.pallas_call(
        _tcomb_body,
        out_shape=jax.ShapeDtypeStruct((_LAYERS, _NCOMB, _D), jnp.float32),
    )(etab, lw, lb)


def _edge_agg_body(x_hbm, idx_hbm, tcomb_hbm, z_hbm, out_hbm,
                   agg_sh, ib0, ib1, db0, db1, xr0, tr0, xr1, tr1,
                   ix0, ix1, gx0, gt0, gx1, gt1, ss0, ss1):
    c = lax.axis_index("c")
    s = lax.axis_index("s")
    wid = c * _SC_TILES + s
    rbase = pl.multiple_of(s * _RPT, 8)

    pltpu.sync_copy(z_hbm, agg_sh.at[pl.ds(rbase, _RPT)])
    plsc.subcore_barrier()

    def start_idx(i, ib, sem):
        pltpu.async_copy(idx_hbm.at[wid, i], ib, sem)

    def wait_idx(ib, sem):
        pltpu.make_async_copy(idx_hbm.at[wid, 0], ib, sem).wait()

    def start_gathers(ib, xr, tr, sx, st):
        pltpu.async_copy(x_hbm.at[ib.at[0]], xr, sx)
        pltpu.async_copy(tcomb_hbm.at[ib.at[1]], tr, st)

    def wait_gathers(xr, tr, sx, st):
        pltpu.make_async_copy(x_hbm.at[ib0.at[0]], xr, sx).wait()
        pltpu.make_async_copy(tcomb_hbm.at[ib0.at[1]], tr, st).wait()

    def start_scatter(db, xr, sem):
        pltpu.async_copy(xr, agg_sh.at[db], sem, add=True)

    def wait_scatter(xr, sem):
        pltpu.make_async_copy(xr, agg_sh.at[db0], sem).wait()

    pad_row = jnp.full((16,), _N, jnp.int32)
    for k in range(_B // 16):
        db1[pl.ds(k * 16, 16)] = pad_row
    pltpu.sync_copy(idx_hbm.at[wid, 0], ib0)
    start_gathers(ib0, xr0, tr0, gx0, gt0)
    start_idx(1, ib1, ix1)
    start_scatter(db1, xr1, ss1)

    def half(j, ib, db, xr, tr, myix, gxs, gts, myss,
             oib, oxr, otr, oix, ogx, ogt, oss):
        wait_gathers(xr, tr, gxs, gts)
        for k in range(_B // 16):
            db[pl.ds(k * 16, 16)] = ib[2, pl.ds(k * 16, 16)]

        @plsc.parallel_loop(0, _B, 1, unroll=8)
        def _edge(k):
            for jj in range(8):
                sl = pl.ds(jj * 16, 16)
                xr[k, sl] = jnp.maximum(xr[k, sl] + tr[k, sl], 0.0)
        start_scatter(db, xr, myss)
        wait_scatter(oxr, oss)
        start_idx(j + 2, ib, myix)
        wait_idx(oib, oix)
        start_gathers(oib, oxr, otr, ogx, ogt)

    def group(g, _):
        half(2 * g, ib0, db0, xr0, tr0, ix0, gx0, gt0, ss0,
             ib1, xr1, tr1, ix1, gx1, gt1, ss1)
        half(2 * g + 1, ib1, db1, xr1, tr1, ix1, gx1, gt1, ss1,
             ib0, xr0, tr0, ix0, gx0, gt0, ss0)
        return 0

    lax.fori_loop(0, _NG, group, 0)
    wait_gathers(xr0, tr0, gx0, gt0)
    wait_idx(ib1, ix1)
    wait_scatter(xr1, ss1)
    plsc.subcore_barrier()
    pltpu.sync_copy(agg_sh.at[pl.ds(rbase, _RPT)],
                    out_hbm.at[c, pl.ds(rbase, _RPT)])


@functools.cache
def _make_edge_agg():
    return pl.kernel(
        _edge_agg_body,
        out_type=jax.ShapeDtypeStruct((_SC_CORES, _NPAD, _D), jnp.float32),
        mesh=plsc.VectorSubcoreMesh(core_axis_name="c", subcore_axis_name="s",
                                    num_cores=_SC_CORES,
                                    num_subcores=_SC_TILES),
        scratch_types=[
            pltpu.VMEM_SHARED((_NPAD, _D), jnp.float32),
            pltpu.VMEM((3, _B), jnp.int32),
            pltpu.VMEM((3, _B), jnp.int32),
            pltpu.VMEM((_B,), jnp.int32),
            pltpu.VMEM((_B,), jnp.int32),
            pltpu.VMEM((_B, _D), jnp.float32),
            pltpu.VMEM((_B, _D), jnp.float32),
            pltpu.VMEM((_B, _D), jnp.float32),
            pltpu.VMEM((_B, _D), jnp.float32),
        ] + [pltpu.SemaphoreType.DMA] * 8,
    )


def _edge_agg(*args):
    return _make_edge_agg()(*args)


def _mlp_body(x_ref, agg_ref, w1_ref, b1_ref, w2_ref, b2_ref, out_ref):
    h = x_ref[...] + agg_ref[0] + agg_ref[1]
    h1 = jnp.maximum(
        jnp.dot(h, w1_ref[...], preferred_element_type=jnp.float32)
        + b1_ref[...], 0.0)
    h2 = (jnp.dot(h1, w2_ref[...], preferred_element_type=jnp.float32)
          + b2_ref[...])
    out_ref[...] = jnp.maximum(h2, 0.0)


def _mlp(x, agg, w1, b1, w2, b2):
    blk = 1000
    return pl.pallas_call(
        _mlp_body,
        grid=(_N // blk,),
        in_specs=[
            pl.BlockSpec((blk, _D), lambda i: (i, 0)),
            pl.BlockSpec((_SC_CORES, blk, _D), lambda i: (0, i, 0)),
            pl.BlockSpec((_D, 2 * _D), lambda i: (0, 0)),
            pl.BlockSpec((1, 2 * _D), lambda i: (0, 0)),
            pl.BlockSpec((2 * _D, _D), lambda i: (0, 0)),
            pl.BlockSpec((1, _D), lambda i: (0, 0)),
        ],
        out_specs=pl.BlockSpec((blk, _D), lambda i: (i, 0)),
        out_shape=jax.ShapeDtypeStruct((_N, _D), jnp.float32),
    )(x, agg, w1, b1, w2, b2)


def _pool_body(x_ref, b_ref, pw1_ref, pb1_ref, pw2_ref, pb2_ref, out_ref,
               g_ref):
    i = pl.program_id(0)

    @pl.when(i == 0)
    def _():
        g_ref[...] = jnp.zeros_like(g_ref)

    oh = (b_ref[...].reshape(-1, 1)
          == lax.broadcasted_iota(jnp.int32, (1, _G), 1)).astype(jnp.float32)
    g_ref[...] += lax.dot_general(
        oh, x_ref[...], (((0,), (0,)), ((), ())),
        preferred_element_type=jnp.float32)

    @pl.when(i == pl.num_programs(0) - 1)
    def _():
        g = g_ref[...]
        h = jnp.maximum(
            jnp.dot(g, pw1_ref[...], preferred_element_type=jnp.float32)
            + pb1_ref[...], 0.0)
        o = (jnp.dot(h, pw2_ref[...], preferred_element_type=jnp.float32)
             + pb2_ref[...])
        nrm = jnp.sqrt(jnp.sum(o * o, axis=-1, keepdims=True))
        out_ref[...] = o / jnp.maximum(nrm, 1e-12)


def _pool_proj(x, batch_r, pw1, pb1, pw2, pb2):
    blk = 1000
    return pl.pallas_call(
        _pool_body,
        grid=(_N // blk,),
        in_specs=[
            pl.BlockSpec((blk, _D), lambda i: (i, 0)),
            pl.BlockSpec((1, 1, blk), lambda i: (i, 0, 0)),
            pl.BlockSpec((_D, _D), lambda i: (0, 0)),
            pl.BlockSpec((1, _D), lambda i: (0, 0)),
            pl.BlockSpec((_D, _OUT), lambda i: (0, 0)),
            pl.BlockSpec((1, _OUT), lambda i: (0, 0)),
        ],
        out_specs=pl.BlockSpec((_G, _OUT), lambda i: (0, 0)),
        out_shape=jax.ShapeDtypeStruct((_G, _OUT), jnp.float32),
        scratch_shapes=[pltpu.VMEM((_G, _D), jnp.float32)],
    )(x, batch_r, pw1, pb1, pw2, pb2)


def kernel(x, edge_index, edge_attr, batch, params):
    node_tab = jnp.concatenate(params['node_emb'], axis=0)
    node_tab = jnp.pad(node_tab, ((0, _NODE_PAD - _NODE_TOT), (0, 0)))
    edge_tab = jnp.concatenate(params['edge_emb'], axis=0)
    edge_tab = jnp.pad(edge_tab, ((0, _EDGE_PAD - _EDGE_TOT), (0, 0)))
    lw = jnp.stack(params['lin_w'])
    lb = jnp.stack(params['lin_b'])

    src = edge_index[0]
    dst = edge_index[1]
    ea_r = edge_attr.T.reshape(3, _E // 128, 128)
    cidx = _compute_cidx(ea_r).reshape(_E)

    zpad = jnp.zeros((_NWORK, _EPAD), jnp.int32)
    dpad = jnp.broadcast_to(
        jnp.arange(_EPAD, dtype=jnp.int32) + _N, (_NWORK, _EPAD))
    srcp = jnp.concatenate([src.reshape(_NWORK, _EPT), zpad], axis=1)
    cidp = jnp.concatenate([cidx.reshape(_NWORK, _EPT), zpad], axis=1)
    dstp = jnp.concatenate([dst.reshape(_NWORK, _EPT), dpad], axis=1)
    idx_all = jnp.stack(
        [a.reshape(_NWORK, _NCHT, _B) for a in (srcp, cidp, dstp)], axis=2)

    tcomb = _build_tcomb(edge_tab, lw, lb)
    h = _node_encode(x, node_tab)

    zeros = jnp.zeros((_RPT, _D), jnp.float32)
    for l in range(_LAYERS):
        agg = _edge_agg(h, idx_all, tcomb[l], zeros)
        h = _mlp(h, agg,
                 params['mlp_w1'][l], params['mlp_b1'][l].reshape(1, -1),
                 params['mlp_w2'][l], params['mlp_b2'][l].reshape(1, -1))

    batch_r = batch.reshape(10, 1, _N // 10)
    return _pool_proj(h, batch_r,
                      params['proj_w1'], params['proj_b1'].reshape(1, -1),
                      params['proj_w2'], params['proj_b2'].reshape(1, -1))


# speedup vs baseline: 4.3741x; 4.3741x over previous
import functools

import jax
import jax.numpy as jnp
from jax import lax
from jax.experimental import pallas as pl
from jax.experimental.pallas import tpu as pltpu
from jax.experimental.pallas import tpu_sc as plsc

_N = 10000
_E = 320000
_D = 128
_G = 256
_OUT = 768
_LAYERS = 4
_NODE_SIZES = [119, 9, 11, 12, 9, 5, 8, 2, 2]
_NODE_TOT = sum(_NODE_SIZES)
_NODE_PAD = 184
_EDGE_SIZES = [22, 6, 2]
_EDGE_TOT = sum(_EDGE_SIZES)
_EDGE_PAD = 32
_NCOMB = 264

_SC_CORES = 2
_SC_TILES = 16
_NWORK = _SC_CORES * _SC_TILES
_EPT = _E // _NWORK
_B = 80
_NCHT = 128
_NCHP = 126
_NG = _NCHP // 2
_EPAD = _NCHT * _B - _EPT
_NPAD = 10240
_RPT = _NPAD // _SC_TILES


def _cidx_body(ea_ref, out_ref):
    a0 = ea_ref[0]
    a1 = ea_ref[1]
    a2 = ea_ref[2]
    out_ref[...] = a0 * 12 + a1 * 2 + a2


def _compute_cidx(ea_r):
    return pl.pallas_call(
        _cidx_body,
        out_shape=jax.ShapeDtypeStruct((_E // 128, 128), jnp.int32),
    )(ea_r)


def _node_enc_body(xc_ref, tab_ref, out_ref):
    cols = lax.broadcasted_iota(jnp.int32, (xc_ref.shape[0], _NODE_PAD), 1)
    mh = jnp.zeros((xc_ref.shape[0], _NODE_PAD), jnp.float32)
    off = 0
    for j, sz in enumerate(_NODE_SIZES):
        idx = xc_ref[:, j][:, None] + off
        mh = mh + (cols == idx).astype(jnp.float32)
        off += sz
    out_ref[...] = jnp.dot(mh, tab_ref[...],
                           preferred_element_type=jnp.float32)


def _node_encode(x_cat, node_tab):
    blk = 1000
    return pl.pallas_call(
        _node_enc_body,
        grid=(_N // blk,),
        in_specs=[
            pl.BlockSpec((blk, 9), lambda i: (i, 0)),
            pl.BlockSpec((_NODE_PAD, _D), lambda i: (0, 0)),
        ],
        out_specs=pl.BlockSpec((blk, _D), lambda i: (i, 0)),
        out_shape=jax.ShapeDtypeStruct((_N, _D), jnp.float32),
    )(x_cat, node_tab)


def _tcomb_body(etab_ref, lw_ref, lb_ref, out_ref):
    rows = lax.broadcasted_iota(jnp.int32, (_NCOMB, _EDGE_PAD), 0)
    cols = lax.broadcasted_iota(jnp.int32, (_NCOMB, _EDGE_PAD), 1)
    c0 = rows // 12
    c1 = (rows % 12) // 2
    c2 = rows % 2
    sel = ((cols == c0) | (cols == c1 + 22) | (cols == c2 + 28))
    comb = jnp.dot(sel.astype(jnp.float32), etab_ref[...],
                   preferred_element_type=jnp.float32)
    for l in range(_LAYERS):
        out_ref[l] = (
            jnp.dot(comb, lw_ref[l], preferred_element_type=jnp.float32)
            + lb_ref[l][None, :]
        )


def _build_tcomb(etab, lw, lb):
    return pl.pallas_call(
        _tcomb_body,
        out_shape=jax.ShapeDtypeStruct((_LAYERS, _NCOMB, _D), jnp.float32),
    )(etab, lw, lb)


def _edge_agg_body(x_hbm, idx_hbm, tcomb_hbm, z_hbm, out_hbm,
                   agg_sh, tc_sh, ib0, ib1, db0, db1, xr0, tr0, xr1, tr1,
                   ix0, ix1, gx0, gt0, gx1, gt1, ss0, ss1):
    c = lax.axis_index("c")
    s = lax.axis_index("s")
    wid = c * _SC_TILES + s
    rbase = pl.multiple_of(s * _RPT, 8)

    pltpu.sync_copy(z_hbm, agg_sh.at[pl.ds(rbase, _RPT)])

    @pl.when(s == 0)
    def _():
        pltpu.sync_copy(tcomb_hbm, tc_sh)

    plsc.subcore_barrier()

    def start_idx(i, ib, sem):
        pltpu.async_copy(idx_hbm.at[wid, i], ib, sem)

    def wait_idx(ib, sem):
        pltpu.make_async_copy(idx_hbm.at[wid, 0], ib, sem).wait()

    def start_gathers(ib, xr, tr, sx, st):
        for u in range(_B // 16):
            sl = pl.ds(u * 16, 16)
            pltpu.async_copy(x_hbm.at[ib.at[0, sl]], xr.at[sl], sx)
        pltpu.async_copy(tc_sh.at[ib.at[1]], tr, st)

    def wait_gathers(xr, tr, sx, st):
        for u in range(_B // 16):
            sl = pl.ds(u * 16, 16)
            pltpu.make_async_copy(x_hbm.at[ib0.at[0, sl]], xr.at[sl],
                                  sx).wait()
        pltpu.make_async_copy(tc_sh.at[ib0.at[1]], tr, st).wait()

    def start_scatter(db, xr, sem):
        pltpu.async_copy(xr, agg_sh.at[db], sem, add=True)

    def wait_scatter(xr, sem):
        pltpu.make_async_copy(xr, agg_sh.at[db0], sem).wait()

    pad_row = jnp.full((16,), _N, jnp.int32)
    for k in range(_B // 16):
        db1[pl.ds(k * 16, 16)] = pad_row
    pltpu.sync_copy(idx_hbm.at[wid, 0], ib0)
    start_gathers(ib0, xr0, tr0, gx0, gt0)
    start_idx(1, ib1, ix1)
    start_scatter(db1, xr1, ss1)

    def half(j, ib, db, xr, tr, myix, gxs, gts, myss,
             oib, oxr, otr, oix, ogx, ogt, oss):
        wait_gathers(xr, tr, gxs, gts)
        wait_idx(oib, oix)
        wait_scatter(oxr, oss)
        start_gathers(oib, oxr, otr, ogx, ogt)
        for k in range(_B // 16):
            db[pl.ds(k * 16, 16)] = ib[2, pl.ds(k * 16, 16)]

        @plsc.parallel_loop(0, _B, 1, unroll=8)
        def _edge(k):
            for jj in range(8):
                sl = pl.ds(jj * 16, 16)
                xr[k, sl] = jnp.maximum(xr[k, sl] + tr[k, sl], 0.0)
        start_scatter(db, xr, myss)
        start_idx(j + 2, ib, myix)

    def group(g, _):
        half(2 * g, ib0, db0, xr0, tr0, ix0, gx0, gt0, ss0,
             ib1, xr1, tr1, ix1, gx1, gt1, ss1)
        half(2 * g + 1, ib1, db1, xr1, tr1, ix1, gx1, gt1, ss1,
             ib0, xr0, tr0, ix0, gx0, gt0, ss0)
        return 0

    lax.fori_loop(0, _NG, group, 0)
    wait_gathers(xr0, tr0, gx0, gt0)
    wait_idx(ib1, ix1)
    wait_scatter(xr1, ss1)
    plsc.subcore_barrier()
    pltpu.sync_copy(agg_sh.at[pl.ds(rbase, _RPT)],
                    out_hbm.at[c, pl.ds(rbase, _RPT)])


@functools.cache
def _make_edge_agg():
    return pl.kernel(
        _edge_agg_body,
        out_type=jax.ShapeDtypeStruct((_SC_CORES, _NPAD, _D), jnp.float32),
        mesh=plsc.VectorSubcoreMesh(core_axis_name="c", subcore_axis_name="s",
                                    num_cores=_SC_CORES,
                                    num_subcores=_SC_TILES),
        scratch_types=[
            pltpu.VMEM_SHARED((_NPAD, _D), jnp.float32),
            pltpu.VMEM_SHARED((_NCOMB, _D), jnp.float32),
            pltpu.VMEM((3, _B), jnp.int32),
            pltpu.VMEM((3, _B), jnp.int32),
            pltpu.VMEM((_B,), jnp.int32),
            pltpu.VMEM((_B,), jnp.int32),
            pltpu.VMEM((_B, _D), jnp.float32),
            pltpu.VMEM((_B, _D), jnp.float32),
            pltpu.VMEM((_B, _D), jnp.float32),
            pltpu.VMEM((_B, _D), jnp.float32),
        ] + [pltpu.SemaphoreType.DMA] * 8,
    )


def _edge_agg(*args):
    return _make_edge_agg()(*args)


def _mlp_body(x_ref, agg_ref, w1_ref, b1_ref, w2_ref, b2_ref, out_ref):
    h = x_ref[...] + agg_ref[0] + agg_ref[1]
    h1 = jnp.maximum(
        jnp.dot(h, w1_ref[...], preferred_element_type=jnp.float32)
        + b1_ref[...], 0.0)
    h2 = (jnp.dot(h1, w2_ref[...], preferred_element_type=jnp.float32)
          + b2_ref[...])
    out_ref[...] = jnp.maximum(h2, 0.0)


def _mlp(x, agg, w1, b1, w2, b2):
    blk = 1000
    return pl.pallas_call(
        _mlp_body,
        grid=(_N // blk,),
        in_specs=[
            pl.BlockSpec((blk, _D), lambda i: (i, 0)),
            pl.BlockSpec((_SC_CORES, blk, _D), lambda i: (0, i, 0)),
            pl.BlockSpec((_D, 2 * _D), lambda i: (0, 0)),
            pl.BlockSpec((1, 2 * _D), lambda i: (0, 0)),
            pl.BlockSpec((2 * _D, _D), lambda i: (0, 0)),
            pl.BlockSpec((1, _D), lambda i: (0, 0)),
        ],
        out_specs=pl.BlockSpec((blk, _D), lambda i: (i, 0)),
        out_shape=jax.ShapeDtypeStruct((_N, _D), jnp.float32),
    )(x, agg, w1, b1, w2, b2)


def _pool_body(x_ref, b_ref, pw1_ref, pb1_ref, pw2_ref, pb2_ref, out_ref,
               g_ref):
    i = pl.program_id(0)

    @pl.when(i == 0)
    def _():
        g_ref[...] = jnp.zeros_like(g_ref)

    oh = (b_ref[...].reshape(-1, 1)
          == lax.broadcasted_iota(jnp.int32, (1, _G), 1)).astype(jnp.float32)
    g_ref[...] += lax.dot_general(
        oh, x_ref[...], (((0,), (0,)), ((), ())),
        preferred_element_type=jnp.float32)

    @pl.when(i == pl.num_programs(0) - 1)
    def _():
        g = g_ref[...]
        h = jnp.maximum(
            jnp.dot(g, pw1_ref[...], preferred_element_type=jnp.float32)
            + pb1_ref[...], 0.0)
        o = (jnp.dot(h, pw2_ref[...], preferred_element_type=jnp.float32)
             + pb2_ref[...])
        nrm = jnp.sqrt(jnp.sum(o * o, axis=-1, keepdims=True))
        out_ref[...] = o / jnp.maximum(nrm, 1e-12)


def _pool_proj(x, batch_r, pw1, pb1, pw2, pb2):
    blk = 1000
    return pl.pallas_call(
        _pool_body,
        grid=(_N // blk,),
        in_specs=[
            pl.BlockSpec((blk, _D), lambda i: (i, 0)),
            pl.BlockSpec((1, 1, blk), lambda i: (i, 0, 0)),
            pl.BlockSpec((_D, _D), lambda i: (0, 0)),
            pl.BlockSpec((1, _D), lambda i: (0, 0)),
            pl.BlockSpec((_D, _OUT), lambda i: (0, 0)),
            pl.BlockSpec((1, _OUT), lambda i: (0, 0)),
        ],
        out_specs=pl.BlockSpec((_G, _OUT), lambda i: (0, 0)),
        out_shape=jax.ShapeDtypeStruct((_G, _OUT), jnp.float32),
        scratch_shapes=[pltpu.VMEM((_G, _D), jnp.float32)],
    )(x, batch_r, pw1, pb1, pw2, pb2)


def kernel(x, edge_index, edge_attr, batch, params):
    node_tab = jnp.concatenate(params['node_emb'], axis=0)
    node_tab = jnp.pad(node_tab, ((0, _NODE_PAD - _NODE_TOT), (0, 0)))
    edge_tab = jnp.concatenate(params['edge_emb'], axis=0)
    edge_tab = jnp.pad(edge_tab, ((0, _EDGE_PAD - _EDGE_TOT), (0, 0)))
    lw = jnp.stack(params['lin_w'])
    lb = jnp.stack(params['lin_b'])

    src = edge_index[0]
    dst = edge_index[1]
    ea_r = edge_attr.T.reshape(3, _E // 128, 128)
    cidx = _compute_cidx(ea_r).reshape(_E)

    zpad = jnp.zeros((_NWORK, _EPAD), jnp.int32)
    dpad = jnp.broadcast_to(
        jnp.arange(_EPAD, dtype=jnp.int32) + _N, (_NWORK, _EPAD))
    srcp = jnp.concatenate([src.reshape(_NWORK, _EPT), zpad], axis=1)
    cidp = jnp.concatenate([cidx.reshape(_NWORK, _EPT), zpad], axis=1)
    dstp = jnp.concatenate([dst.reshape(_NWORK, _EPT), dpad], axis=1)
    idx_all = jnp.stack(
        [a.reshape(_NWORK, _NCHT, _B) for a in (srcp, cidp, dstp)], axis=2)

    tcomb = _build_tcomb(edge_tab, lw, lb)
    h = _node_encode(x, node_tab)

    zeros = jnp.zeros((_RPT, _D), jnp.float32)
    for l in range(_LAYERS):
        agg = _edge_agg(h, idx_all, tcomb[l], zeros)
        h = _mlp(h, agg,
                 params['mlp_w1'][l], params['mlp_b1'][l].reshape(1, -1),
                 params['mlp_w2'][l], params['mlp_b2'][l].reshape(1, -1))

    batch_r = batch.reshape(10, 1, _N // 10)
    return _pool_proj(h, batch_r,
                      params['proj_w1'], params['proj_b1'].reshape(1, -1),
                      params['proj_w2'], params['proj_b2'].reshape(1, -1))


# speedup vs baseline: 4.3761x; 1.0004x over previous
import functools

import jax
import jax.numpy as jnp
from jax import lax
from jax.experimental import pallas as pl
from jax.experimental.pallas import tpu as pltpu
from jax.experimental.pallas import tpu_sc as plsc

_N = 10000
_E = 320000
_D = 128
_G = 256
_OUT = 768
_LAYERS = 4
_NODE_SIZES = [119, 9, 11, 12, 9, 5, 8, 2, 2]
_NODE_TOT = sum(_NODE_SIZES)
_NODE_PAD = 184
_EDGE_SIZES = [22, 6, 2]
_EDGE_TOT = sum(_EDGE_SIZES)
_EDGE_PAD = 32
_NCOMB = 264

_SC_CORES = 2
_SC_TILES = 16
_NWORK = _SC_CORES * _SC_TILES
_EPT = _E // _NWORK
_B = 80
_NCHT = 128
_NCHP = 126
_NG = _NCHP // 2
_EPAD = _NCHT * _B - _EPT
_NPAD = 10240
_RPT = _NPAD // _SC_TILES


def _cidx_body(ea_ref, out_ref):
    a0 = ea_ref[0]
    a1 = ea_ref[1]
    a2 = ea_ref[2]
    out_ref[...] = a0 * 12 + a1 * 2 + a2


def _compute_cidx(ea_r):
    return pl.pallas_call(
        _cidx_body,
        out_shape=jax.ShapeDtypeStruct((_E // 128, 128), jnp.int32),
    )(ea_r)


def _node_enc_body(xc_ref, tab_ref, out_ref):
    cols = lax.broadcasted_iota(jnp.int32, (xc_ref.shape[0], _NODE_PAD), 1)
    mh = jnp.zeros((xc_ref.shape[0], _NODE_PAD), jnp.float32)
    off = 0
    for j, sz in enumerate(_NODE_SIZES):
        idx = xc_ref[:, j][:, None] + off
        mh = mh + (cols == idx).astype(jnp.float32)
        off += sz
    out_ref[...] = jnp.dot(mh, tab_ref[...],
                           preferred_element_type=jnp.float32)


def _node_encode(x_cat, node_tab):
    blk = 1000
    return pl.pallas_call(
        _node_enc_body,
        grid=(_N // blk,),
        in_specs=[
            pl.BlockSpec((blk, 9), lambda i: (i, 0)),
            pl.BlockSpec((_NODE_PAD, _D), lambda i: (0, 0)),
        ],
        out_specs=pl.BlockSpec((blk, _D), lambda i: (i, 0)),
        out_shape=jax.ShapeDtypeStruct((_N, _D), jnp.float32),
    )(x_cat, node_tab)


def _tcomb_body(etab_ref, lw_ref, lb_ref, out_ref):
    rows = lax.broadcasted_iota(jnp.int32, (_NCOMB, _EDGE_PAD), 0)
    cols = lax.broadcasted_iota(jnp.int32, (_NCOMB, _EDGE_PAD), 1)
    c0 = rows // 12
    c1 = (rows % 12) // 2
    c2 = rows % 2
    sel = ((cols == c0) | (cols == c1 + 22) | (cols == c2 + 28))
    comb = jnp.dot(sel.astype(jnp.float32), etab_ref[...],
                   preferred_element_type=jnp.float32)
    for l in range(_LAYERS):
        out_ref[l] = (
            jnp.dot(comb, lw_ref[l], preferred_element_type=jnp.float32)
            + lb_ref[l][None, :]
        )


def _build_tcomb(etab, lw, lb):
    return pl.pallas_call(
        _tcomb_body,
        out_shape=jax.ShapeDtypeStruct((_LAYERS, _NCOMB, _D), jnp.float32),
    )(etab, lw, lb)


def _edge_agg_body(x_hbm, idx_hbm, tcomb_hbm, z_hbm, out_hbm,
                   agg_sh, tc_sh, ib0, ib1, db0, db1, xr0, tr0, xr1, tr1,
                   ix0, ix1, gx0, gt0, gx1, gt1, ss0, ss1):
    c = lax.axis_index("c")
    s = lax.axis_index("s")
    wid = c * _SC_TILES + s
    rbase = pl.multiple_of(s * _RPT, 8)

    pltpu.sync_copy(z_hbm, agg_sh.at[pl.ds(rbase, _RPT)])

    @pl.when(s == 0)
    def _():
        pltpu.sync_copy(tcomb_hbm, tc_sh)

    plsc.subcore_barrier()

    def start_idx(i, ib, sem):
        pltpu.async_copy(idx_hbm.at[wid, i], ib, sem)

    def wait_idx(ib, sem):
        pltpu.make_async_copy(idx_hbm.at[wid, 0], ib, sem).wait()

    def start_gathers(ib, xr, tr, sx, st):
        for u in range(_B // 8):
            sl = pl.ds(u * 8, 8)
            pltpu.async_copy(x_hbm.at[ib.at[0, sl]], xr.at[sl], sx)
        pltpu.async_copy(tc_sh.at[ib.at[1]], tr, st)

    def wait_gathers(xr, tr, sx, st):
        for u in range(_B // 8):
            sl = pl.ds(u * 8, 8)
            pltpu.make_async_copy(x_hbm.at[ib0.at[0, sl]], xr.at[sl],
                                  sx).wait()
        pltpu.make_async_copy(tc_sh.at[ib0.at[1]], tr, st).wait()

    def start_scatter(db, xr, sem):
        pltpu.async_copy(xr, agg_sh.at[db], sem, add=True)

    def wait_scatter(xr, sem):
        pltpu.make_async_copy(xr, agg_sh.at[db0], sem).wait()

    pad_row = jnp.full((16,), _N, jnp.int32)
    for k in range(_B // 16):
        db1[pl.ds(k * 16, 16)] = pad_row
    pltpu.sync_copy(idx_hbm.at[wid, 0], ib0)
    start_gathers(ib0, xr0, tr0, gx0, gt0)
    start_idx(1, ib1, ix1)
    start_scatter(db1, xr1, ss1)

    def half(j, ib, db, xr, tr, myix, gxs, gts, myss,
             oib, oxr, otr, oix, ogx, ogt, oss):
        wait_gathers(xr, tr, gxs, gts)
        wait_idx(oib, oix)
        wait_scatter(oxr, oss)
        start_gathers(oib, oxr, otr, ogx, ogt)
        for k in range(_B // 16):
            db[pl.ds(k * 16, 16)] = ib[2, pl.ds(k * 16, 16)]

        @plsc.parallel_loop(0, _B, 1, unroll=8)
        def _edge(k):
            for jj in range(8):
                sl = pl.ds(jj * 16, 16)
                xr[k, sl] = jnp.maximum(xr[k, sl] + tr[k, sl], 0.0)
        start_scatter(db, xr, myss)
        start_idx(j + 2, ib, myix)

    def group(g, _):
        half(2 * g, ib0, db0, xr0, tr0, ix0, gx0, gt0, ss0,
             ib1, xr1, tr1, ix1, gx1, gt1, ss1)
        half(2 * g + 1, ib1, db1, xr1, tr1, ix1, gx1, gt1, ss1,
             ib0, xr0, tr0, ix0, gx0, gt0, ss0)
        return 0

    lax.fori_loop(0, _NG, group, 0)
    wait_gathers(xr0, tr0, gx0, gt0)
    wait_idx(ib1, ix1)
    wait_scatter(xr1, ss1)
    plsc.subcore_barrier()
    pltpu.sync_copy(agg_sh.at[pl.ds(rbase, _RPT)],
                    out_hbm.at[c, pl.ds(rbase, _RPT)])


@functools.cache
def _make_edge_agg():
    return pl.kernel(
        _edge_agg_body,
        out_type=jax.ShapeDtypeStruct((_SC_CORES, _NPAD, _D), jnp.float32),
        mesh=plsc.VectorSubcoreMesh(core_axis_name="c", subcore_axis_name="s",
                                    num_cores=_SC_CORES,
                                    num_subcores=_SC_TILES),
        scratch_types=[
            pltpu.VMEM_SHARED((_NPAD, _D), jnp.float32),
            pltpu.VMEM_SHARED((_NCOMB, _D), jnp.float32),
            pltpu.VMEM((3, _B), jnp.int32),
            pltpu.VMEM((3, _B), jnp.int32),
            pltpu.VMEM((_B,), jnp.int32),
            pltpu.VMEM((_B,), jnp.int32),
            pltpu.VMEM((_B, _D), jnp.float32),
            pltpu.VMEM((_B, _D), jnp.float32),
            pltpu.VMEM((_B, _D), jnp.float32),
            pltpu.VMEM((_B, _D), jnp.float32),
        ] + [pltpu.SemaphoreType.DMA] * 8,
    )


def _edge_agg(*args):
    return _make_edge_agg()(*args)


def _mlp_body(x_ref, agg_ref, w1_ref, b1_ref, w2_ref, b2_ref, out_ref):
    h = x_ref[...] + agg_ref[0] + agg_ref[1]
    h1 = jnp.maximum(
        jnp.dot(h, w1_ref[...], preferred_element_type=jnp.float32)
        + b1_ref[...], 0.0)
    h2 = (jnp.dot(h1, w2_ref[...], preferred_element_type=jnp.float32)
          + b2_ref[...])
    out_ref[...] = jnp.maximum(h2, 0.0)


def _mlp(x, agg, w1, b1, w2, b2):
    blk = 1000
    return pl.pallas_call(
        _mlp_body,
        grid=(_N // blk,),
        in_specs=[
            pl.BlockSpec((blk, _D), lambda i: (i, 0)),
            pl.BlockSpec((_SC_CORES, blk, _D), lambda i: (0, i, 0)),
            pl.BlockSpec((_D, 2 * _D), lambda i: (0, 0)),
            pl.BlockSpec((1, 2 * _D), lambda i: (0, 0)),
            pl.BlockSpec((2 * _D, _D), lambda i: (0, 0)),
            pl.BlockSpec((1, _D), lambda i: (0, 0)),
        ],
        out_specs=pl.BlockSpec((blk, _D), lambda i: (i, 0)),
        out_shape=jax.ShapeDtypeStruct((_N, _D), jnp.float32),
    )(x, agg, w1, b1, w2, b2)


def _pool_body(x_ref, b_ref, pw1_ref, pb1_ref, pw2_ref, pb2_ref, out_ref,
               g_ref):
    i = pl.program_id(0)

    @pl.when(i == 0)
    def _():
        g_ref[...] = jnp.zeros_like(g_ref)

    oh = (b_ref[...].reshape(-1, 1)
          == lax.broadcasted_iota(jnp.int32, (1, _G), 1)).astype(jnp.float32)
    g_ref[...] += lax.dot_general(
        oh, x_ref[...], (((0,), (0,)), ((), ())),
        preferred_element_type=jnp.float32)

    @pl.when(i == pl.num_programs(0) - 1)
    def _():
        g = g_ref[...]
        h = jnp.maximum(
            jnp.dot(g, pw1_ref[...], preferred_element_type=jnp.float32)
            + pb1_ref[...], 0.0)
        o = (jnp.dot(h, pw2_ref[...], preferred_element_type=jnp.float32)
             + pb2_ref[...])
        nrm = jnp.sqrt(jnp.sum(o * o, axis=-1, keepdims=True))
        out_ref[...] = o / jnp.maximum(nrm, 1e-12)


def _pool_proj(x, batch_r, pw1, pb1, pw2, pb2):
    blk = 1000
    return pl.pallas_call(
        _pool_body,
        grid=(_N // blk,),
        in_specs=[
            pl.BlockSpec((blk, _D), lambda i: (i, 0)),
            pl.BlockSpec((1, 1, blk), lambda i: (i, 0, 0)),
            pl.BlockSpec((_D, _D), lambda i: (0, 0)),
            pl.BlockSpec((1, _D), lambda i: (0, 0)),
            pl.BlockSpec((_D, _OUT), lambda i: (0, 0)),
            pl.BlockSpec((1, _OUT), lambda i: (0, 0)),
        ],
        out_specs=pl.BlockSpec((_G, _OUT), lambda i: (0, 0)),
        out_shape=jax.ShapeDtypeStruct((_G, _OUT), jnp.float32),
        scratch_shapes=[pltpu.VMEM((_G, _D), jnp.float32)],
    )(x, batch_r, pw1, pb1, pw2, pb2)


def kernel(x, edge_index, edge_attr, batch, params):
    node_tab = jnp.concatenate(params['node_emb'], axis=0)
    node_tab = jnp.pad(node_tab, ((0, _NODE_PAD - _NODE_TOT), (0, 0)))
    edge_tab = jnp.concatenate(params['edge_emb'], axis=0)
    edge_tab = jnp.pad(edge_tab, ((0, _EDGE_PAD - _EDGE_TOT), (0, 0)))
    lw = jnp.stack(params['lin_w'])
    lb = jnp.stack(params['lin_b'])

    src = edge_index[0]
    dst = edge_index[1]
    ea_r = edge_attr.T.reshape(3, _E // 128, 128)
    cidx = _compute_cidx(ea_r).reshape(_E)

    zpad = jnp.zeros((_NWORK, _EPAD), jnp.int32)
    dpad = jnp.broadcast_to(
        jnp.arange(_EPAD, dtype=jnp.int32) + _N, (_NWORK, _EPAD))
    srcp = jnp.concatenate([src.reshape(_NWORK, _EPT), zpad], axis=1)
    cidp = jnp.concatenate([cidx.reshape(_NWORK, _EPT), zpad], axis=1)
    dstp = jnp.concatenate([dst.reshape(_NWORK, _EPT), dpad], axis=1)
    idx_all = jnp.stack(
        [a.reshape(_NWORK, _NCHT, _B) for a in (srcp, cidp, dstp)], axis=2)

    tcomb = _build_tcomb(edge_tab, lw, lb)
    h = _node_encode(x, node_tab)

    zeros = jnp.zeros((_RPT, _D), jnp.float32)
    for l in range(_LAYERS):
        agg = _edge_agg(h, idx_all, tcomb[l], zeros)
        h = _mlp(h, agg,
                 params['mlp_w1'][l], params['mlp_b1'][l].reshape(1, -1),
                 params['mlp_w2'][l], params['mlp_b2'][l].reshape(1, -1))

    batch_r = batch.reshape(10, 1, _N // 10)
    return _pool_proj(h, batch_r,
                      params['proj_w1'], params['proj_b1'].reshape(1, -1),
                      params['proj_w2'], params['proj_b2'].reshape(1, -1))
